# baseline (speedup 1.0000x reference)
import functools

import jax
import jax.numpy as jnp
from jax import lax
from jax.experimental import pallas as pl
from jax.experimental.pallas import tpu as pltpu
from jax.experimental.pallas import tpu_sc as plsc


@functools.lru_cache(maxsize=None)
def _build(R, T, max_blocks):
    spec_num = T - 1
    info = plsc.get_sparse_core_info()
    NC, NS, L = info.num_cores, info.num_subcores, info.num_lanes
    NW = NC * NS
    rows_per_w = R // NW
    elems_per_w = rows_per_w * T
    n_vec = elems_per_w // L
    off_acc = R
    off_samp = 2 * R
    off_spec = 2 * R + R * T
    off_bs = 2 * R + R * T + R * spec_num
    mesh = plsc.VectorSubcoreMesh(core_axis_name="c", subcore_axis_name="s")

    @functools.partial(
        pl.kernel,
        mesh=mesh,
        compiler_params=pltpu.CompilerParams(needs_layout_passes=False),
        out_type=[jax.ShapeDtypeStruct((R * T,), jnp.int32)] * 4,
        scratch_types=[
            pltpu.VMEM((L,), jnp.int32),
            pltpu.VMEM((rows_per_w,), jnp.int32),
            pltpu.VMEM((rows_per_w,), jnp.int32),
            pltpu.VMEM((T * rows_per_w,), jnp.int32),
            pltpu.VMEM((spec_num * rows_per_w,), jnp.int32),
            pltpu.VMEM((rows_per_w, max_blocks), jnp.int32),
            pltpu.VMEM((elems_per_w,), jnp.int32),
            pltpu.VMEM((elems_per_w,), jnp.int32),
            pltpu.VMEM((elems_per_w,), jnp.int32),
            pltpu.VMEM((elems_per_w,), jnp.int32),
            pltpu.SemaphoreType.DMA,
        ],
    )
    def body(packed_hbm, bt_hbm, tok_hbm, opos_hbm, olen_hbm, oslot_hbm,
             bs_v, pos_v, acc_v, samp_v, spec_v, bt_v,
             tok_o, pos_o, len_o, slot_o, sem):
        wid = lax.axis_index("s") * NC + lax.axis_index("c")
        r0 = wid * rows_per_w
        copies = [
            pltpu.async_copy(packed_hbm.at[pl.ds(off_bs, L)], bs_v, sem),
            pltpu.async_copy(packed_hbm.at[pl.ds(r0, rows_per_w)], pos_v, sem),
            pltpu.async_copy(
                packed_hbm.at[pl.ds(off_acc + r0, rows_per_w)], acc_v, sem),
            pltpu.async_copy(
                packed_hbm.at[pl.ds(off_samp + r0 * T, T * rows_per_w)], samp_v, sem),
            pltpu.async_copy(
                packed_hbm.at[pl.ds(off_spec + r0 * spec_num,
                                    spec_num * rows_per_w)], spec_v, sem),
            pltpu.async_copy(bt_hbm.at[pl.ds(r0, rows_per_w)], bt_v, sem),
        ]
        for c in copies:
            c.wait()
        lane = lax.iota(jnp.int32, L)
        bs = bs_v[...]
        one = jnp.int32(1)
        for k in range(n_vec):
            f = lane + jnp.int32(k * L)
            i_loc = lax.div(f, jnp.int32(T))
            j = f - i_loc * jnp.int32(T)
            acc = plsc.load_gather(acc_v, [i_loc])
            base = plsc.load_gather(pos_v, [i_loc]) + acc
            position = base + j
            blk_col = lax.div(position, bs)
            blk = plsc.load_gather(bt_v, [i_loc, blk_col])
            slot = blk * bs + (position - blk_col * bs)
            tok_bonus = plsc.load_gather(samp_v, [i_loc * jnp.int32(T) + acc - one])
            tok_spec = plsc.load_gather(
                spec_v, [i_loc * jnp.int32(spec_num) + jnp.maximum(j - one, 0)])
            tok = jnp.where(j == 0, tok_bonus, tok_spec)
            sl = pl.ds(k * L, L)
            tok_o[sl] = tok
            pos_o[sl] = position
            len_o[sl] = position + one
            slot_o[sl] = slot
        e0 = wid * elems_per_w
        out_copies = [
            pltpu.async_copy(tok_o, tok_hbm.at[pl.ds(e0, elems_per_w)], sem),
            pltpu.async_copy(pos_o, opos_hbm.at[pl.ds(e0, elems_per_w)], sem),
            pltpu.async_copy(len_o, olen_hbm.at[pl.ds(e0, elems_per_w)], sem),
            pltpu.async_copy(slot_o, oslot_hbm.at[pl.ds(e0, elems_per_w)], sem),
        ]
        for c in out_copies:
            c.wait()

    return body


def kernel(input_tokens, sampled_tokens, input_positions, seq_lens, slot_mapping,
           block_table, spec_tokens, accepted_num, num_seqs, num_queries, block_size):
    R = sampled_tokens.shape[0]
    spec_num = spec_tokens.shape[1]
    T = 1 + spec_num
    max_blocks = block_table.shape[1]
    i64 = input_positions.dtype
    fn = _build(R, T, max_blocks)
    packed = jnp.concatenate([
        input_positions.astype(jnp.int32),
        accepted_num.astype(jnp.int32),
        sampled_tokens.astype(jnp.int32).reshape(-1),
        spec_tokens.astype(jnp.int32).reshape(-1),
        jnp.full((16,), block_size, jnp.int32),
    ])
    tok, pos, slen, slot = fn(packed, block_table)
    return (tok.astype(i64), pos.astype(i64), slen.astype(i64), slot.astype(i64))
